# baseline (speedup 1.0000x reference)
import functools

import jax
import jax.numpy as jnp
from jax import lax
from jax.experimental import pallas as pl
from jax.experimental.pallas import tpu as pltpu
from jax.experimental.pallas import tpu_sc as plsc

N = 100000
D = 128
DIMG = 85
NG = 18
NP = 7
NFUSED = NG * NP

C = 128
NFULL = N // C
REM = N - NFULL * C
NCHUNK = NFULL + (1 if REM else 0)
NPAD = NCHUNK * C

NC = 2
NS = 16
NW = NC * NS
L = 16
KMAX = -(-NCHUNK // NW)
REM_WID = NFULL % NW

_mesh = plsc.VectorSubcoreMesh(core_axis_name="c", subcore_axis_name="s")


@functools.partial(
    pl.kernel,
    out_type=jax.ShapeDtypeStruct((N, D), jnp.float32),
    mesh=_mesh,
    scratch_types=[
        pltpu.VMEM_SHARED((128, D), jnp.float32),
        pltpu.VMEM((NG, D), jnp.float32),
        pltpu.VMEM((NP, D), jnp.float32),
        pltpu.VMEM((8, D), jnp.float32),
        [pltpu.VMEM((C,), jnp.int32)] * 3,
        [pltpu.VMEM((C,), jnp.int32)] * 3,
        [pltpu.VMEM((C,), jnp.int32)] * 3,
        [pltpu.VMEM((C, D), jnp.float32)] * 3,
        [pltpu.SemaphoreType.DMA] * 3,
        [pltpu.SemaphoreType.DMA] * 3,
        [pltpu.SemaphoreType.DMA] * 3,
    ],
)
def _sc_embed(gpad_hbm, ppad_hbm, group_hbm, period_hbm, node_hbm,
              fused_sp, gpad_v, ppad_v, frows_v, gidx, pidx, fidx, rows,
              sidx, sgat, swri):
    cid = lax.axis_index("c")
    sid = lax.axis_index("s")
    wid = sid * NC + cid

    pltpu.sync_copy(gpad_hbm, gpad_v)
    pltpu.sync_copy(ppad_hbm, ppad_v)
    for j in range(8):
        r = sid * 8 + j

        @pl.when(r < NFUSED)
        def _():
            g = r // NP
            p = r - g * NP
            for q in range(D // L):
                sl = pl.ds(q * L, L)
                frows_v[j, sl] = gpad_v[g, sl] + ppad_v[p, sl]

    pltpu.sync_copy(frows_v, fused_sp.at[pl.ds(sid * 8, 8)])
    plsc.subcore_barrier()


    def chunk_of(k):
        return wid + NW * k

    def issue_idx(k, b):
        base = chunk_of(k) * C
        pltpu.async_copy(group_hbm.at[pl.ds(base, C)], gidx[b], sidx[b])
        pltpu.async_copy(period_hbm.at[pl.ds(base, C)], pidx[b], sidx[b])

    def wait_idx(k, b):
        base = chunk_of(k) * C
        pltpu.make_async_copy(group_hbm.at[pl.ds(base, C)], gidx[b],
                              sidx[b]).wait()
        pltpu.make_async_copy(period_hbm.at[pl.ds(base, C)], pidx[b],
                              sidx[b]).wait()

    def wait_write(k, b):
        base = chunk_of(k) * C
        pltpu.make_async_copy(rows[b], node_hbm.at[pl.ds(base, C)],
                              swri[b]).wait()

    def wait_gather_issue_write(k, b):
        base = chunk_of(k) * C
        pltpu.make_async_copy(fused_sp.at[fidx[b]], rows[b], sgat[b]).wait()
        pltpu.async_copy(rows[b], node_hbm.at[pl.ds(base, C)], swri[b])

    NSLOT = 3
    for k in range(NSLOT):
        issue_idx(k, k)

    for k in range(KMAX):
        b = k % NSLOT

        if k >= NSLOT:
            @pl.when(chunk_of(k - NSLOT) < NFULL)
            def _():
                wait_write(k - NSLOT, b)

        @pl.when(chunk_of(k) < NFULL)
        def _():
            wait_idx(k, b)
            for q in range(C // L):
                sl = pl.ds(q * L, L)
                fidx[b][sl] = gidx[b][sl] * NP + pidx[b][sl]

            if k + NSLOT < KMAX:
                @pl.when(chunk_of(k + NSLOT) < NFULL)
                def _():
                    issue_idx(k + NSLOT, b)

            pltpu.async_copy(fused_sp.at[fidx[b]], rows[b], sgat[b])

        if k >= 1:
            @pl.when(chunk_of(k - 1) < NFULL)
            def _():
                wait_gather_issue_write(k - 1, (k - 1) % NSLOT)

    @pl.when(chunk_of(KMAX - 1) < NFULL)
    def _():
        wait_gather_issue_write(KMAX - 1, (KMAX - 1) % NSLOT)

    for k in range(KMAX - NSLOT, KMAX):
        @pl.when(chunk_of(k) < NFULL)
        def _():
            wait_write(k, k % NSLOT)

    if REM:
        @pl.when(wid == REM_WID)
        def _():
            base = NFULL * C
            pltpu.sync_copy(group_hbm.at[pl.ds(base, C)], gidx[0])
            pltpu.sync_copy(period_hbm.at[pl.ds(base, C)], pidx[0])
            for q in range(C // L):
                sl = pl.ds(q * L, L)
                fidx[0][sl] = gidx[0][sl] * NP + pidx[0][sl]
            pltpu.async_copy(fused_sp.at[fidx[0]], rows[0], sgat[0]).wait()
            pltpu.sync_copy(rows[0].at[pl.ds(0, REM)],
                            node_hbm.at[pl.ds(base, REM)])


def kernel(group, period, gtable, ptable):
    gpad = jnp.pad(gtable, ((0, 0), (0, D - DIMG)))
    ppad = jnp.pad(ptable, ((0, 0), (DIMG, 0)))
    gidx = jnp.pad(group.astype(jnp.int32), (0, NPAD - N))
    pidx = jnp.pad(period.astype(jnp.int32), (0, NPAD - N))
    node = _sc_embed(gpad, ppad, gidx, pidx)
    return node


# speedup vs baseline: 1.0804x; 1.0804x over previous
import functools

import jax
import jax.numpy as jnp
from jax import lax
from jax.experimental import pallas as pl
from jax.experimental.pallas import tpu as pltpu
from jax.experimental.pallas import tpu_sc as plsc

N = 100000
D = 128
DIMG = 85
NG = 18
NP = 7
NFUSED = NG * NP

C = 128
NFULL = N // C
REM = N - NFULL * C

NC = 2
NS = 16
NW = NC * NS
L = 16
KMAX = -(-NFULL // NW)
REM_WID = NFULL % NW

_mesh = plsc.VectorSubcoreMesh(core_axis_name="c", subcore_axis_name="s")


@functools.partial(
    pl.kernel,
    out_type=jax.ShapeDtypeStruct((N, D), jnp.float32),
    mesh=_mesh,
    scratch_types=[
        pltpu.VMEM_SHARED((128, D), jnp.float32),
        pltpu.VMEM((NG, D), jnp.float32),
        pltpu.VMEM((NP, D), jnp.float32),
        pltpu.VMEM((8, D), jnp.float32),
        [pltpu.VMEM((C,), jnp.int32)] * 2,
        [pltpu.VMEM((C,), jnp.int32)] * 2,
        [pltpu.VMEM((C,), jnp.int32)] * 2,
        pltpu.VMEM((REM,), jnp.int32),
        [pltpu.VMEM((C, D), jnp.float32)] * 2,
        [pltpu.SemaphoreType.DMA] * 2,
        [pltpu.SemaphoreType.DMA] * 2,
        [pltpu.SemaphoreType.DMA] * 2,
    ],
)
def _sc_embed(gpad_hbm, ppad_hbm, group_hbm, period_hbm, node_hbm,
              fused_sp, gpad_v, ppad_v, frows_v, gidx, pidx, fidx, fidx_rem,
              rows, sidx, sgat, swri):
    cid = lax.axis_index("c")
    sid = lax.axis_index("s")
    wid = sid * NC + cid

    pltpu.sync_copy(gpad_hbm, gpad_v)
    pltpu.sync_copy(ppad_hbm, ppad_v)
    for j in range(8):
        r = sid * 8 + j

        @pl.when(r < NFUSED)
        def _():
            g = r // NP
            p = r - g * NP
            for q in range(D // L):
                sl = pl.ds(q * L, L)
                frows_v[j, sl] = gpad_v[g, sl] + ppad_v[p, sl]

    pltpu.sync_copy(frows_v, fused_sp.at[pl.ds(sid * 8, 8)])
    plsc.subcore_barrier()


    def chunk_of(k):
        return wid + NW * k

    def issue_idx(k, b):
        base = chunk_of(k) * C
        pltpu.async_copy(group_hbm.at[pl.ds(base, C)], gidx[b], sidx[b])
        pltpu.async_copy(period_hbm.at[pl.ds(base, C)], pidx[b], sidx[b])

    def wait_idx(k, b):
        base = chunk_of(k) * C
        pltpu.make_async_copy(group_hbm.at[pl.ds(base, C)], gidx[b],
                              sidx[b]).wait()
        pltpu.make_async_copy(period_hbm.at[pl.ds(base, C)], pidx[b],
                              sidx[b]).wait()

    def wait_write(k, b):
        base = chunk_of(k) * C
        pltpu.make_async_copy(rows[b], node_hbm.at[pl.ds(base, C)],
                              swri[b]).wait()

    issue_idx(0, 0)
    issue_idx(1, 1)

    for k in range(KMAX):
        b = k % 2

        if k >= 2:
            @pl.when(chunk_of(k - 2) < NFULL)
            def _():
                wait_write(k - 2, b)

        @pl.when(chunk_of(k) < NFULL)
        def _():
            wait_idx(k, b)
            for q in range(C // L):
                sl = pl.ds(q * L, L)
                fidx[b][sl] = gidx[b][sl] * NP + pidx[b][sl]

            if k + 2 < KMAX:
                @pl.when(chunk_of(k + 2) < NFULL)
                def _():
                    issue_idx(k + 2, b)

            base = chunk_of(k) * C
            pltpu.async_copy(fused_sp.at[fidx[b]], rows[b], sgat[b]).wait()
            pltpu.async_copy(rows[b], node_hbm.at[pl.ds(base, C)], swri[b])

    for k in (KMAX - 2, KMAX - 1):
        @pl.when(chunk_of(k) < NFULL)
        def _():
            wait_write(k, k % 2)

    if REM:
        @pl.when(wid == REM_WID)
        def _():
            base = NFULL * C
            rsl = pl.ds(0, REM)
            pltpu.sync_copy(group_hbm.at[pl.ds(base, REM)], gidx[0].at[rsl])
            pltpu.sync_copy(period_hbm.at[pl.ds(base, REM)], pidx[0].at[rsl])
            for q in range(REM // L):
                sl = pl.ds(q * L, L)
                fidx_rem[sl] = gidx[0][sl] * NP + pidx[0][sl]
            pltpu.async_copy(fused_sp.at[fidx_rem], rows[0].at[rsl],
                             sgat[0]).wait()
            pltpu.sync_copy(rows[0].at[rsl], node_hbm.at[pl.ds(base, REM)])


def kernel(group, period, gtable, ptable):
    gpad = jnp.pad(gtable, ((0, 0), (0, D - DIMG)))
    ppad = jnp.pad(ptable, ((0, 0), (DIMG, 0)))
    return _sc_embed(gpad, ppad, group.astype(jnp.int32),
                     period.astype(jnp.int32))
